# baseline (speedup 1.0000x reference)
import functools

import jax
import jax.numpy as jnp
from jax import lax
from jax.experimental import pallas as pl
from jax.experimental.pallas import tpu as pltpu
from jax.experimental.pallas import tpu_sc as plsc

B, L = 4096, 50
N = B * L
D = 32
OUTD = 3 * D
NW = 32
BB = B // NW


def _make_kernel():
    info = plsc.get_sparse_core_info()
    nc = info.num_cores
    mesh = plsc.VectorSubcoreMesh(core_axis_name="c", subcore_axis_name="s")

    @functools.partial(
        pl.kernel,
        mesh=mesh,
        out_type=jax.ShapeDtypeStruct((L, OUTD // 8, NW, 8, BB), jnp.float32),
        compiler_params=pltpu.CompilerParams(use_tc_tiling_on_sc=False,
                                             needs_layout_passes=False),
        scratch_types=[
            pltpu.VMEM((BB * L,), jnp.int32),
            pltpu.VMEM((BB * L,), jnp.int32),
            pltpu.VMEM((BB * L,), jnp.int32),
            pltpu.VMEM((L, BB), jnp.int32),
            pltpu.VMEM((L, BB), jnp.int32),
            pltpu.VMEM((L, BB), jnp.int32),
            pltpu.VMEM((BB, D), jnp.float32),
            pltpu.VMEM((BB, D), jnp.float32),
            pltpu.VMEM((BB, D), jnp.float32),
            pltpu.VMEM((BB, D), jnp.float32),
            pltpu.VMEM((BB, D), jnp.float32),
            pltpu.VMEM((BB, D), jnp.float32),
            pltpu.VMEM((OUTD // 8, 8, BB), jnp.float32),
            pltpu.VMEM((OUTD // 8, 8, BB), jnp.float32),
            pltpu.SemaphoreType.DMA,
            pltpu.SemaphoreType.DMA,
        ],
    )
    def k(idx0_h, idx1_h, idx2_h, emb0_h, emb1_h, emb2_h, out_h,
          iv0, iv1, iv2, ivT0, ivT1, ivT2,
          g00, g01, g02, g10, g11, g12, t0, t1, gsem, wsem):
        wid = lax.axis_index("s") * nc + lax.axis_index("c")
        base = wid * (BB * L)

        ivs = (iv0, iv1, iv2)
        ivTs = (ivT0, ivT1, ivT2)
        embs = (emb0_h, emb1_h, emb2_h)
        gbufs = ((g00, g01, g02), (g10, g11, g12))
        tbufs = (t0, t1)

        hs = [pltpu.async_copy(ih.at[pl.ds(base, BB * L)], iv, gsem)
              for ih, iv in zip((idx0_h, idx1_h, idx2_h), ivs)]
        for h in hs:
            h.wait()

        lane = lax.broadcasted_iota(jnp.int32, (16,), 0)
        laneL = lane * L
        rows = [g * 16 + lane for g in range(BB // 16)]

        def idx_t_body(l, carry):
            for f in range(3):
                for g in range(BB // 16):
                    v = plsc.load_gather(ivs[f], [laneL + (g * 16 * L + l)])
                    ivTs[f][l, pl.ds(g * 16, 16)] = v
            return carry

        lax.fori_loop(0, L, idx_t_body, 0)

        def fire_gather(l, par):
            for f in range(3):
                pltpu.async_copy(embs[f].at[ivTs[f].at[l]],
                                 gbufs[par][f], gsem)

        def wait_gather(par):
            for f in range(3):
                pltpu.make_async_copy(embs[f].at[ivTs[f].at[0]],
                                      gbufs[par][f], gsem).wait()

        def transpose(par):
            def tr_body(d, carry):
                r = d // 8
                s = d - r * 8
                cols = jnp.zeros((16,), jnp.int32) + d
                for f in range(3):
                    for g in range(BB // 16):
                        v = plsc.load_gather(gbufs[par][f], [rows[g], cols])
                        tbufs[par][f * 4 + r, s, pl.ds(g * 16, 16)] = v
                return carry

            lax.fori_loop(0, D, tr_body, 0)

        def fire_write(l, par):
            pltpu.async_copy(tbufs[par], out_h.at[l, :, wid], wsem)

        def wait_write(par):
            pltpu.make_async_copy(tbufs[par], out_h.at[0, :, wid],
                                  wsem).wait()

        fire_gather(0, 0)
        fire_gather(1, 1)
        wait_gather(0)
        transpose(0)
        fire_write(0, 0)
        fire_gather(2, 0)
        wait_gather(1)
        transpose(1)
        fire_write(1, 1)

        def body(gi, carry):
            for bpar in range(2):
                l = 2 + gi * 2 + bpar
                par = bpar
                pltpu.async_copy(embs[0].at[ivTs[0].at[l + 1]],
                                 gbufs[1 - par][0], gsem)
                pltpu.async_copy(embs[1].at[ivTs[1].at[l + 1]],
                                 gbufs[1 - par][1], gsem)
                pltpu.async_copy(embs[2].at[ivTs[2].at[l + 1]],
                                 gbufs[1 - par][2], gsem)
                wait_write(par)
                wait_gather(par)
                transpose(par)
                fire_write(l, par)
            return carry

        lax.fori_loop(0, (L - 4) // 2, body, 0)

        fire_gather(L - 1, 1)
        wait_write(0)
        wait_gather(0)
        transpose(0)
        fire_write(L - 2, 0)
        wait_write(1)
        wait_gather(1)
        transpose(1)
        fire_write(L - 1, 1)
        wait_write(0)
        wait_write(1)

    return k


_kern = _make_kernel()


def kernel(idx0, idx1, idx2, emb0, emb1, emb2):
    out5 = _kern(idx0.reshape(N), idx1.reshape(N), idx2.reshape(N),
                 emb0, emb1, emb2)
    return jnp.transpose(out5, (2, 4, 0, 1, 3)).reshape(B, L, OUTD)
